# baseline (speedup 1.0000x reference)
import functools

import jax
import jax.numpy as jnp
from jax import lax
from jax.experimental import pallas as pl
from jax.experimental.pallas import tpu as pltpu
from jax.experimental.pallas import tpu_sc as plsc

EPS = 1e-6
THETA = 10000.0
NEG = -1e30
HD = 128
BF16 = jnp.bfloat16


def _dot(a, b):
    return jnp.dot(a.astype(BF16), b.astype(BF16), preferred_element_type=F32)
F32 = jnp.float32


def _pick(n, pref):
    b = pref
    while n % b:
        b -= 128
    return b


def _rms(x, w):
    return x * jax.lax.rsqrt(jnp.mean(x * x, axis=-1, keepdims=True) + EPS) * w


def _qkv_body(x_ref, ln_ref, w_ref, b_ref, cos_ref, sin_ref, o_ref, *, bn, kv_end):
    x = x_ref[...]
    xn = _rms(x, ln_ref[...])
    o = _dot(xn, w_ref[...]) + b_ref[...]
    n = pl.program_id(1)
    bm = o.shape[0]
    oh = o.reshape(bm, bn // HD, HD)
    o1 = oh[:, :, : HD // 2]
    o2 = oh[:, :, HD // 2 :]
    cos = cos_ref[...][:, None, :]
    sin = sin_ref[...][:, None, :]
    roped = (oh * cos + jnp.concatenate([-o2, o1], axis=-1) * sin).reshape(bm, bn)
    o_ref[...] = jnp.where(n < kv_end, roped, o)


def _qkv(x, ln1, wqkv, bqkv, cos, sin, kv_cols, bm, bn):
    T, D = x.shape
    N = wqkv.shape[1]
    body = functools.partial(_qkv_body, bn=bn, kv_end=kv_cols // bn)
    return pl.pallas_call(
        body,
        grid=(T // bm, N // bn),
        in_specs=[
            pl.BlockSpec((bm, D), lambda m, n: (m, 0)),
            pl.BlockSpec((1, D), lambda m, n: (0, 0)),
            pl.BlockSpec((D, bn), lambda m, n: (0, n)),
            pl.BlockSpec((1, bn), lambda m, n: (0, n)),
            pl.BlockSpec((bm, HD), lambda m, n: (m, 0)),
            pl.BlockSpec((bm, HD), lambda m, n: (m, 0)),
        ],
        out_specs=pl.BlockSpec((bm, bn), lambda m, n: (m, n)),
        out_shape=jax.ShapeDtypeStruct((T, N), F32),
    )(x, ln1.reshape(1, D), wqkv, bqkv.reshape(1, N), cos, sin)


def _attn_body(q_ref, k_ref, v_ref, o_ref, *, bq, scale):
    q = q_ref[...]
    k = k_ref[...]
    s = jax.lax.dot_general(q.astype(BF16), k.astype(BF16), (((1,), (1,)), ((), ())),
                            preferred_element_type=F32) * scale
    Tk = k.shape[0]
    row = pl.program_id(1) * bq + jax.lax.broadcasted_iota(jnp.int32, (bq, Tk), 0)
    col = jax.lax.broadcasted_iota(jnp.int32, (bq, Tk), 1)
    s = jnp.where(row >= col, s, NEG)
    m = jnp.max(s, axis=-1, keepdims=True)
    p = jnp.exp(s - m)
    l = jnp.sum(p, axis=-1, keepdims=True)
    o = _dot(p, v_ref[...])
    o_ref[...] = o / l


def _attention(qkv, T, H, NKV, bq):
    scale = HD ** -0.5
    body = functools.partial(_attn_body, bq=bq, scale=scale)
    return pl.pallas_call(
        body,
        grid=(H, T // bq),
        in_specs=[
            pl.BlockSpec((bq, HD), lambda h, mq: (mq, h)),
            pl.BlockSpec((T, HD), lambda h, mq: (0, H + h)),
            pl.BlockSpec((T, HD), lambda h, mq: (0, H + NKV + h)),
        ],
        out_specs=pl.BlockSpec((bq, HD), lambda h, mq: (mq, h)),
        out_shape=jax.ShapeDtypeStruct((T, H * HD), F32),
    )(qkv, qkv, qkv)


def _wo_body(a_ref, w_ref, r_ref, o_ref):
    o_ref[...] = r_ref[...] + _dot(a_ref[...], w_ref[...])


def _wo_proj(attn, wo, resid, bm, bn):
    T, Din = attn.shape
    D = wo.shape[1]
    return pl.pallas_call(
        _wo_body,
        grid=(T // bm, D // bn),
        in_specs=[
            pl.BlockSpec((bm, Din), lambda m, n: (m, 0)),
            pl.BlockSpec((Din, bn), lambda m, n: (0, n)),
            pl.BlockSpec((bm, bn), lambda m, n: (m, n)),
        ],
        out_specs=pl.BlockSpec((bm, bn), lambda m, n: (m, n)),
        out_shape=jax.ShapeDtypeStruct((T, D), F32),
    )(attn, wo, resid)


def _router_body(h_ref, ln_ref, g_ref, x_ref, wi_ref, wf_ref, *, E):
    h = h_ref[...]
    xn = _rms(h, ln_ref[...])
    x_ref[...] = xn
    lg = _dot(xn, g_ref[...])
    bm, C = lg.shape
    cols = jax.lax.broadcasted_iota(jnp.int32, (bm, C), 1)
    rl = jnp.where(cols < E, lg, NEG)
    mx = jnp.max(rl, axis=-1, keepdims=True)
    ex = jnp.exp(rl - mx)
    pr = ex / jnp.sum(ex, axis=-1, keepdims=True)
    v1 = jnp.max(pr, axis=-1, keepdims=True)
    i1 = jnp.min(jnp.where(pr == v1, cols, C), axis=-1, keepdims=True)
    pr2 = jnp.where(cols == i1, -1.0, pr)
    v2 = jnp.max(pr2, axis=-1, keepdims=True)
    i2 = jnp.min(jnp.where(pr2 == v2, cols, C), axis=-1, keepdims=True)
    wsum = v1 + v2
    sgl = jnp.sum(jnp.where(cols == E, lg, 0.0), axis=-1, keepdims=True)
    sg = jax.nn.sigmoid(sgl)
    wi_ref[...] = jnp.where(cols == 0, i1, jnp.where(cols == 1, i2, 0))
    wf_ref[...] = jnp.where(cols == 0, v1 / wsum,
                            jnp.where(cols == 1, v2 / wsum,
                                      jnp.where(cols == 2, sg, 0.0)))


def _router(h1, ln2, gpad, E, bm):
    T, D = h1.shape
    C = gpad.shape[1]
    body = functools.partial(_router_body, E=E)
    return pl.pallas_call(
        body,
        grid=(T // bm,),
        in_specs=[
            pl.BlockSpec((bm, D), lambda m: (m, 0)),
            pl.BlockSpec((1, D), lambda m: (0, 0)),
            pl.BlockSpec((D, C), lambda m: (0, 0)),
        ],
        out_specs=[
            pl.BlockSpec((bm, D), lambda m: (m, 0)),
            pl.BlockSpec((bm, C), lambda m: (m, 0)),
            pl.BlockSpec((bm, C), lambda m: (m, 0)),
        ],
        out_shape=[
            jax.ShapeDtypeStruct((T, D), F32),
            jax.ShapeDtypeStruct((T, C), jnp.int32),
            jax.ShapeDtypeStruct((T, C), F32),
        ],
    )(h1, ln2.reshape(1, D), gpad)


def _w1_body(te_ref, xs_ref, wg_ref, wu_ref, o_ref):
    x = xs_ref[...]
    D, FF = wg_ref.shape[1], wg_ref.shape[2]
    wg = wg_ref[...].reshape(D, FF)
    wu = wu_ref[...].reshape(D, FF)
    g = _dot(x, wg)
    u = _dot(x, wu)
    o_ref[...] = (g * jax.nn.sigmoid(g)) * u


def _moe_w1(xs, w1, te, bm):
    P, D = xs.shape
    E = w1.shape[0]
    FF = w1.shape[2] // 2
    grid_spec = pltpu.PrefetchScalarGridSpec(
        num_scalar_prefetch=1,
        grid=(P // bm,),
        in_specs=[
            pl.BlockSpec((bm, D), lambda m, te: (m, 0)),
            pl.BlockSpec((1, D, FF), lambda m, te: (te[m], 0, 0)),
            pl.BlockSpec((1, D, FF), lambda m, te: (te[m], 0, 1)),
        ],
        out_specs=pl.BlockSpec((bm, FF), lambda m, te: (m, 0)),
    )
    return pl.pallas_call(
        _w1_body,
        grid_spec=grid_spec,
        out_shape=jax.ShapeDtypeStruct((P, FF), F32),
    )(te, xs, w1, w1)


def _w2_body(te_ref, a_ref, w_ref, ws_ref, o_ref):
    a = a_ref[...]
    FF, D = w_ref.shape[1], w_ref.shape[2]
    w = w_ref[...].reshape(FF, D)
    y = _dot(a, w)
    o_ref[...] = y * ws_ref[...]


def _moe_w2(act, w2, ws, te, bm):
    P, FF = act.shape
    E, _, D = w2.shape
    grid_spec = pltpu.PrefetchScalarGridSpec(
        num_scalar_prefetch=1,
        grid=(P // bm,),
        in_specs=[
            pl.BlockSpec((bm, FF), lambda m, te: (m, 0)),
            pl.BlockSpec((1, FF, D), lambda m, te: (te[m], 0, 0)),
            pl.BlockSpec((bm, 1), lambda m, te: (m, 0)),
        ],
        out_specs=pl.BlockSpec((bm, D), lambda m, te: (m, 0)),
    )
    return pl.pallas_call(
        _w2_body,
        grid_spec=grid_spec,
        out_shape=jax.ShapeDtypeStruct((P, D), F32),
    )(te, act, w2, ws.reshape(P, 1))


SC_L = 16


def _sc_meta(ea, wa, T, E, P, BMD, NTE):
    A = ea.shape[0]
    NCH = A // SC_L
    mesh = plsc.VectorSubcoreMesh(core_axis_name="c", subcore_axis_name="s")

    @functools.partial(
        pl.kernel, mesh=mesh,
        compiler_params=pltpu.CompilerParams(needs_layout_passes=False),
        out_type=[
            jax.ShapeDtypeStruct((A,), jnp.int32),
            jax.ShapeDtypeStruct((P,), jnp.int32),
            jax.ShapeDtypeStruct((P,), jnp.float32),
            jax.ShapeDtypeStruct((NTE,), jnp.int32),
        ],
        scratch_types=[
            pltpu.VMEM((A,), jnp.int32),
            pltpu.VMEM((A,), jnp.float32),
            pltpu.VMEM((A,), jnp.int32),
            pltpu.VMEM((P,), jnp.int32),
            pltpu.VMEM((P,), jnp.float32),
            pltpu.VMEM((SC_L,), jnp.int32),
            pltpu.VMEM((NTE,), jnp.int32),
        ],
    )
    def k(ea_hbm, wa_hbm, pos_hbm, tok_hbm, ws_hbm, te_hbm,
          ea_v, wa_v, pos_v, tok_v, ws_v, cnt_v, te_v):
        wid = lax.axis_index("s") * 2 + lax.axis_index("c")

        @pl.when(wid == 0)
        def _():
            pltpu.sync_copy(ea_hbm, ea_v)
            pltpu.sync_copy(wa_hbm, wa_v)
            lanes = lax.iota(jnp.int32, SC_L)
            zi = jnp.zeros((SC_L,), jnp.int32)
            zf = jnp.zeros((SC_L,), jnp.float32)

            def zinit(i, _):
                tok_v[pl.ds(i * SC_L, SC_L)] = zi
                ws_v[pl.ds(i * SC_L, SC_L)] = zf
                return 0
            lax.fori_loop(0, P // SC_L, zinit, 0)

            cnt_v[...] = zi

            def count_body(c, _):
                ev = ea_v[pl.ds(c * SC_L, SC_L)]
                cnt = cnt_v[...]
                for e in range(E):
                    n = jnp.sum(jnp.where(ev == e, 1, 0))
                    cnt = cnt + jnp.where(lanes == e, n, 0)
                cnt_v[...] = cnt
                return 0
            lax.fori_loop(0, NCH, count_body, 0)

            tot = cnt_v[...]
            padded = (tot + (BMD - 1)) & (-BMD)
            cum = jnp.cumsum(padded)
            offs = cum - padded
            for v in range(NTE // SC_L):
                tiles = (v * SC_L + lanes) * BMD
                tev = zi
                for e in range(E):
                    cum_e = jnp.sum(jnp.where(lanes == e, cum, 0))
                    tev = tev + jnp.where(tiles >= cum_e, 1, 0)
                te_v[pl.ds(v * SC_L, SC_L)] = jnp.minimum(tev, E - 1)
            cnt_v[...] = offs

            def pos_body(c, _):
                av = c * SC_L + lanes
                ev = ea_v[pl.ds(c * SC_L, SC_L)]
                cnt = cnt_v[...]
                pos = zi
                newcnt = cnt
                for e in range(E):
                    m = ev == e
                    base = jnp.sum(jnp.where(lanes == e, cnt, 0))
                    r = jnp.cumsum(m.astype(jnp.int32))
                    pos = jnp.where(m, base + r - 1, pos)
                    n = jnp.max(r)
                    newcnt = newcnt + jnp.where(lanes == e, n, 0)
                cnt_v[...] = newcnt
                pos_v[pl.ds(c * SC_L, SC_L)] = pos
                tok = jnp.where(av >= T, av - T, av)
                plsc.store_scatter(tok_v, [pos], tok)
                plsc.store_scatter(ws_v, [pos], wa_v[pl.ds(c * SC_L, SC_L)])
                return 0
            lax.fori_loop(0, NCH, pos_body, 0)

            pltpu.sync_copy(pos_v, pos_hbm)
            pltpu.sync_copy(tok_v, tok_hbm)
            pltpu.sync_copy(ws_v, ws_hbm)
            pltpu.sync_copy(te_v, te_hbm)

    return k(ea, wa)


def _sc_gather(x, tok, P, CH=16):
    T, D = x.shape
    mesh = plsc.VectorSubcoreMesh(core_axis_name="c", subcore_axis_name="s")
    NW = 32
    per_w = P // NW

    @functools.partial(
        pl.kernel, mesh=mesh,
        compiler_params=pltpu.CompilerParams(needs_layout_passes=False),
        out_type=jax.ShapeDtypeStruct((P, D), jnp.float32),
        scratch_types=[
            pltpu.VMEM((CH,), jnp.int32),
            pltpu.VMEM((CH,), jnp.int32),
            pltpu.VMEM((CH, D), jnp.float32),
            pltpu.VMEM((CH, D), jnp.float32),
            pltpu.SemaphoreType.DMA,
            pltpu.SemaphoreType.DMA,
        ],
    )
    def k(x_hbm, tok_hbm, xs_hbm, idx0, idx1, buf0, buf1, sem0, sem1):
        wid = lax.axis_index("s") * 2 + lax.axis_index("c")
        base = wid * per_w
        idxs = (idx0, idx1)
        bufs = (buf0, buf1)
        sems = (sem0, sem1)
        n = per_w // CH
        pltpu.sync_copy(tok_hbm.at[pl.ds(base, CH)], idx0)
        cps = [pltpu.async_copy(x_hbm.at[idx0], buf0, sem0)]
        for c in range(n):
            b = c % 2
            nb = (c + 1) % 2
            if c + 1 < n:
                pltpu.sync_copy(tok_hbm.at[pl.ds(base + (c + 1) * CH, CH)], idxs[nb])
                cps.append(pltpu.async_copy(x_hbm.at[idxs[nb]], bufs[nb], sems[nb]))
            cps[c].wait()
            pltpu.sync_copy(bufs[b], xs_hbm.at[pl.ds(base + c * CH, CH)])

    return k(x, tok)


def _sc_combine(y, pos, T, CH=16):
    P, D = y.shape
    mesh = plsc.VectorSubcoreMesh(core_axis_name="c", subcore_axis_name="s")
    NW = 32
    per_w = T // NW

    @functools.partial(
        pl.kernel, mesh=mesh,
        compiler_params=pltpu.CompilerParams(needs_layout_passes=False),
        out_type=[
            jax.ShapeDtypeStruct((T, D), jnp.float32),
            jax.ShapeDtypeStruct((T, D), jnp.float32),
        ],
        scratch_types=[
            pltpu.VMEM((CH,), jnp.int32),
            pltpu.VMEM((CH,), jnp.int32),
            pltpu.VMEM((CH, D), jnp.float32),
            pltpu.VMEM((CH, D), jnp.float32),
            pltpu.SemaphoreType.DMA,
            pltpu.SemaphoreType.DMA,
        ],
    )
    def k(y_hbm, pos_hbm, m0_hbm, m1_hbm, idx0, idx1, buf0, buf1, sem0, sem1):
        wid = lax.axis_index("s") * 2 + lax.axis_index("c")
        base = wid * per_w
        nch = per_w // CH
        idxs = (idx0, idx1)
        bufs = (buf0, buf1)
        sems = (sem0, sem1)
        outs = (m0_hbm, m1_hbm)

        def src(i):
            kk, c = divmod(i, nch)
            return pl.ds(kk * T + base + c * CH, CH)

        def dst(i):
            kk, c = divmod(i, nch)
            return outs[kk].at[pl.ds(base + c * CH, CH)]

        n = 2 * nch
        pltpu.sync_copy(pos_hbm.at[src(0)], idx0)
        cps = [pltpu.async_copy(y_hbm.at[idx0], buf0, sem0)]
        for i in range(n):
            b = i % 2
            nb = (i + 1) % 2
            if i + 1 < n:
                pltpu.sync_copy(pos_hbm.at[src(i + 1)], idxs[nb])
                cps.append(pltpu.async_copy(y_hbm.at[idxs[nb]], bufs[nb], sems[nb]))
            cps[i].wait()
            pltpu.sync_copy(bufs[b], dst(i))

    return k(y, pos)


def _s1_body(x_ref, wg_ref, wu_ref, o_ref):
    x = x_ref[...]
    g = _dot(x, wg_ref[...])
    u = _dot(x, wu_ref[...])
    o_ref[...] = (g * jax.nn.sigmoid(g)) * u


def _shared1(x, sw1, bm, bn, s0, nb):
    T, D = x.shape
    SFF = sw1.shape[1] // 2
    nsh = SFF // bn
    return pl.pallas_call(
        _s1_body,
        grid=(nb, T // bm),
        in_specs=[
            pl.BlockSpec((bm, D), lambda n, m: (m, 0)),
            pl.BlockSpec((D, bn), lambda n, m: (0, s0 + n)),
            pl.BlockSpec((D, bn), lambda n, m: (0, nsh + s0 + n)),
        ],
        out_specs=pl.BlockSpec((bm, bn), lambda n, m: (m, n)),
        out_shape=jax.ShapeDtypeStruct((T, nb * bn), F32),
    )(x, sw1, sw1)


def _s2_body(ha_ref, hb_ref, wa_ref, wb_ref, h1_ref, m0_ref, m1_ref,
             sg_ref, o_ref):
    y = _dot(ha_ref[...], wa_ref[...]) + _dot(hb_ref[...], wb_ref[...])
    o_ref[...] = h1_ref[...] + m0_ref[...] + m1_ref[...] + sg_ref[...] * y


def _shared2(hsa, hsb, sw2, h1, m0, m1, sg, bm, bn):
    T, SFFA = hsa.shape
    D = sw2.shape[1]
    return pl.pallas_call(
        _s2_body,
        grid=(D // bn, T // bm),
        in_specs=[
            pl.BlockSpec((bm, SFFA), lambda n, m: (m, 0)),
            pl.BlockSpec((bm, SFFA), lambda n, m: (m, 0)),
            pl.BlockSpec((SFFA, bn), lambda n, m: (0, n)),
            pl.BlockSpec((SFFA, bn), lambda n, m: (1, n)),
            pl.BlockSpec((bm, bn), lambda n, m: (m, n)),
            pl.BlockSpec((bm, bn), lambda n, m: (m, n)),
            pl.BlockSpec((bm, bn), lambda n, m: (m, n)),
            pl.BlockSpec((bm, 1), lambda n, m: (m, 0)),
        ],
        out_specs=pl.BlockSpec((bm, bn), lambda n, m: (m, n)),
        out_shape=jax.ShapeDtypeStruct((T, D), F32),
    )(hsa, hsb, sw2, sw2, h1, m0, m1, sg)


def kernel(hidden_states, positions, ln1_w, ln2_w, wqkv, bqkv, wo, gate_w, w1, w2, sw1, sw2, sgate_w):
    T, D = hidden_states.shape
    QKVN = wqkv.shape[1]
    H = wo.shape[0] // HD
    NKV = (QKVN // HD - H) // 2
    E = gate_w.shape[1]
    FF = w2.shape[1]
    SFF = sw2.shape[0]
    K = 2
    A = T * K
    BMD = 128
    P = A + E * BMD
    NT = P // BMD

    inv_freq = 1.0 / (THETA ** (jnp.arange(0, HD, 2, dtype=F32) / HD))
    freqs = positions.astype(F32)[:, None] * inv_freq
    cos = jnp.concatenate([jnp.cos(freqs)] * 2, axis=-1)
    sin = jnp.concatenate([jnp.sin(freqs)] * 2, axis=-1)

    bmr = min(256, T)
    kv_cols = (H + NKV) * HD
    import math
    bnq = _pick(math.gcd(QKVN, kv_cols), 512)
    qkv = _qkv(hidden_states, ln1_w, wqkv, bqkv, cos, sin,
               kv_cols=kv_cols, bm=bmr, bn=bnq)
    attn = _attention(qkv, T, H, NKV, bq=bmr)
    h1 = _wo_proj(attn, wo, hidden_states, bm=bmr, bn=_pick(D, 512))

    gpad = jnp.concatenate(
        [gate_w, sgate_w, jnp.zeros((D, 128 - E - 1), F32)], axis=1)
    x, wi, wf = _router(h1, ln2_w, gpad, E, bm=bmr)
    topi = wi[:, :K]
    topv = wf[:, :K]
    sg = wf[:, 2:3]

    ea = jnp.concatenate([wi[:, 0], wi[:, 1]])
    wa = jnp.concatenate([wf[:, 0], wf[:, 1]])
    NTE = ((NT + SC_L - 1) // SC_L) * SC_L
    pos, tok, ws, te_pad = _sc_meta(ea, wa, T, E, P, BMD, NTE)
    te = te_pad[:NT]
    xs = _sc_gather(x, tok, P)

    SFH = SFF // 2
    bns = _pick(SFH, 512)
    nbh = SFH // bns
    hsa = _shared1(x, sw1, bm=bmr, bn=bns, s0=0, nb=nbh)

    act = _moe_w1(xs, w1, te, bm=BMD)
    y = _moe_w2(act, w2, ws, te, bm=BMD)
    m0, m1 = _sc_combine(y, pos, T)
    hsb = _shared1(x, sw1, bm=bmr, bn=bns, s0=nbh, nb=nbh)

    out = _shared2(hsa, hsb, sw2, h1, m0, m1, sg, bm=bmr, bn=_pick(D, 512))
    return out


# speedup vs baseline: 1.1543x; 1.1543x over previous
import functools

import jax
import jax.numpy as jnp
from jax import lax
from jax.experimental import pallas as pl
from jax.experimental.pallas import tpu as pltpu
from jax.experimental.pallas import tpu_sc as plsc

EPS = 1e-6
THETA = 10000.0
NEG = -1e30
HD = 128
BF16 = jnp.bfloat16


def _dot(a, b):
    return jnp.dot(a.astype(BF16), b.astype(BF16), preferred_element_type=F32)
F32 = jnp.float32


def _pick(n, pref):
    b = pref
    while n % b:
        b -= 128
    return b


def _rms(x, w):
    return x * jax.lax.rsqrt(jnp.mean(x * x, axis=-1, keepdims=True) + EPS) * w


def _qkv_body(x_ref, ln_ref, w_ref, b_ref, cos_ref, sin_ref, o_ref, *, bn, kv_end):
    x = x_ref[...]
    xn = _rms(x, ln_ref[...])
    o = _dot(xn, w_ref[...]) + b_ref[...]
    n = pl.program_id(1)
    bm = o.shape[0]
    oh = o.reshape(bm, bn // HD, HD)
    o1 = oh[:, :, : HD // 2]
    o2 = oh[:, :, HD // 2 :]
    cos = cos_ref[...][:, None, :]
    sin = sin_ref[...][:, None, :]
    roped = (oh * cos + jnp.concatenate([-o2, o1], axis=-1) * sin).reshape(bm, bn)
    o_ref[...] = jnp.where(n < kv_end, roped, o)


def _qkv(x, ln1, wqkv, bqkv, cos, sin, kv_cols, bm, bn):
    T, D = x.shape
    N = wqkv.shape[1]
    body = functools.partial(_qkv_body, bn=bn, kv_end=kv_cols // bn)
    return pl.pallas_call(
        body,
        grid=(T // bm, N // bn),
        in_specs=[
            pl.BlockSpec((bm, D), lambda m, n: (m, 0)),
            pl.BlockSpec((1, D), lambda m, n: (0, 0)),
            pl.BlockSpec((D, bn), lambda m, n: (0, n)),
            pl.BlockSpec((1, bn), lambda m, n: (0, n)),
            pl.BlockSpec((bm, HD), lambda m, n: (m, 0)),
            pl.BlockSpec((bm, HD), lambda m, n: (m, 0)),
        ],
        out_specs=pl.BlockSpec((bm, bn), lambda m, n: (m, n)),
        out_shape=jax.ShapeDtypeStruct((T, N), F32),
    )(x, ln1.reshape(1, D), wqkv, bqkv.reshape(1, N), cos, sin)


def _attn_body(q_ref, k_ref, v_ref, o_ref, *, bq, scale):
    q = q_ref[...]
    k = k_ref[...]
    s = jax.lax.dot_general(q.astype(BF16), k.astype(BF16), (((1,), (1,)), ((), ())),
                            preferred_element_type=F32) * scale
    Tk = k.shape[0]
    row = pl.program_id(1) * bq + jax.lax.broadcasted_iota(jnp.int32, (bq, Tk), 0)
    col = jax.lax.broadcasted_iota(jnp.int32, (bq, Tk), 1)
    s = jnp.where(row >= col, s, NEG)
    m = jnp.max(s, axis=-1, keepdims=True)
    p = jnp.exp(s - m)
    l = jnp.sum(p, axis=-1, keepdims=True)
    o = _dot(p, v_ref[...])
    o_ref[...] = o / l


def _attention(qkv, T, H, NKV, bq):
    scale = HD ** -0.5
    body = functools.partial(_attn_body, bq=bq, scale=scale)
    return pl.pallas_call(
        body,
        grid=(H, T // bq),
        in_specs=[
            pl.BlockSpec((bq, HD), lambda h, mq: (mq, h)),
            pl.BlockSpec((T, HD), lambda h, mq: (0, H + h)),
            pl.BlockSpec((T, HD), lambda h, mq: (0, H + NKV + h)),
        ],
        out_specs=pl.BlockSpec((bq, HD), lambda h, mq: (mq, h)),
        out_shape=jax.ShapeDtypeStruct((T, H * HD), F32),
    )(qkv, qkv, qkv)


def _wo_body(a_ref, w_ref, r_ref, o_ref):
    o_ref[...] = r_ref[...] + _dot(a_ref[...], w_ref[...])


def _wo_proj(attn, wo, resid, bm, bn):
    T, Din = attn.shape
    D = wo.shape[1]
    return pl.pallas_call(
        _wo_body,
        grid=(T // bm, D // bn),
        in_specs=[
            pl.BlockSpec((bm, Din), lambda m, n: (m, 0)),
            pl.BlockSpec((Din, bn), lambda m, n: (0, n)),
            pl.BlockSpec((bm, bn), lambda m, n: (m, n)),
        ],
        out_specs=pl.BlockSpec((bm, bn), lambda m, n: (m, n)),
        out_shape=jax.ShapeDtypeStruct((T, D), F32),
    )(attn, wo, resid)


def _router_body(h_ref, ln_ref, g_ref, x_ref, wi_ref, wf_ref, *, E):
    h = h_ref[...]
    xn = _rms(h, ln_ref[...])
    x_ref[...] = xn
    lg = _dot(xn, g_ref[...])
    bm, C = lg.shape
    cols = jax.lax.broadcasted_iota(jnp.int32, (bm, C), 1)
    rl = jnp.where(cols < E, lg, NEG)
    mx = jnp.max(rl, axis=-1, keepdims=True)
    ex = jnp.exp(rl - mx)
    pr = ex / jnp.sum(ex, axis=-1, keepdims=True)
    v1 = jnp.max(pr, axis=-1, keepdims=True)
    i1 = jnp.min(jnp.where(pr == v1, cols, C), axis=-1, keepdims=True)
    pr2 = jnp.where(cols == i1, -1.0, pr)
    v2 = jnp.max(pr2, axis=-1, keepdims=True)
    i2 = jnp.min(jnp.where(pr2 == v2, cols, C), axis=-1, keepdims=True)
    wsum = v1 + v2
    sgl = jnp.sum(jnp.where(cols == E, lg, 0.0), axis=-1, keepdims=True)
    sg = jax.nn.sigmoid(sgl)
    wi_ref[...] = jnp.where(cols == 0, i1, jnp.where(cols == 1, i2, 0))
    wf_ref[...] = jnp.where(cols == 0, v1 / wsum,
                            jnp.where(cols == 1, v2 / wsum,
                                      jnp.where(cols == 2, sg, 0.0)))


def _router(h1, ln2, gpad, E, bm):
    T, D = h1.shape
    C = gpad.shape[1]
    body = functools.partial(_router_body, E=E)
    return pl.pallas_call(
        body,
        grid=(T // bm,),
        in_specs=[
            pl.BlockSpec((bm, D), lambda m: (m, 0)),
            pl.BlockSpec((1, D), lambda m: (0, 0)),
            pl.BlockSpec((D, C), lambda m: (0, 0)),
        ],
        out_specs=[
            pl.BlockSpec((bm, D), lambda m: (m, 0)),
            pl.BlockSpec((bm, C), lambda m: (m, 0)),
            pl.BlockSpec((bm, C), lambda m: (m, 0)),
        ],
        out_shape=[
            jax.ShapeDtypeStruct((T, D), F32),
            jax.ShapeDtypeStruct((T, C), jnp.int32),
            jax.ShapeDtypeStruct((T, C), F32),
        ],
    )(h1, ln2.reshape(1, D), gpad)


def _w1_body(te_ref, xs_ref, wg_ref, wu_ref, o_ref):
    x = xs_ref[...]
    D, FF = wg_ref.shape[1], wg_ref.shape[2]
    wg = wg_ref[...].reshape(D, FF)
    wu = wu_ref[...].reshape(D, FF)
    g = _dot(x, wg)
    u = _dot(x, wu)
    o_ref[...] = (g * jax.nn.sigmoid(g)) * u


def _moe_w1(xs, w1, te, bm):
    P, D = xs.shape
    E = w1.shape[0]
    FF = w1.shape[2] // 2
    grid_spec = pltpu.PrefetchScalarGridSpec(
        num_scalar_prefetch=1,
        grid=(P // bm,),
        in_specs=[
            pl.BlockSpec((bm, D), lambda m, te: (m, 0)),
            pl.BlockSpec((1, D, FF), lambda m, te: (te[m], 0, 0)),
            pl.BlockSpec((1, D, FF), lambda m, te: (te[m], 0, 1)),
        ],
        out_specs=pl.BlockSpec((bm, FF), lambda m, te: (m, 0)),
    )
    return pl.pallas_call(
        _w1_body,
        grid_spec=grid_spec,
        out_shape=jax.ShapeDtypeStruct((P, FF), F32),
    )(te, xs, w1, w1)


def _w2_body(te_ref, a_ref, w_ref, ws_ref, o_ref):
    a = a_ref[...]
    FF, D = w_ref.shape[1], w_ref.shape[2]
    w = w_ref[...].reshape(FF, D)
    y = _dot(a, w)
    o_ref[...] = y * ws_ref[...]


def _moe_w2(act, w2, ws, te, bm):
    P, FF = act.shape
    E, _, D = w2.shape
    grid_spec = pltpu.PrefetchScalarGridSpec(
        num_scalar_prefetch=1,
        grid=(P // bm,),
        in_specs=[
            pl.BlockSpec((bm, FF), lambda m, te: (m, 0)),
            pl.BlockSpec((1, FF, D), lambda m, te: (te[m], 0, 0)),
            pl.BlockSpec((bm, 1), lambda m, te: (m, 0)),
        ],
        out_specs=pl.BlockSpec((bm, D), lambda m, te: (m, 0)),
    )
    return pl.pallas_call(
        _w2_body,
        grid_spec=grid_spec,
        out_shape=jax.ShapeDtypeStruct((P, D), F32),
    )(te, act, w2, ws.reshape(P, 1))


SC_L = 16


def _sc_meta(ea, wa, T, E, P, BMD, NTE):
    A = ea.shape[0]
    NCH = A // SC_L
    mesh = plsc.VectorSubcoreMesh(core_axis_name="c", subcore_axis_name="s")

    @functools.partial(
        pl.kernel, mesh=mesh,
        compiler_params=pltpu.CompilerParams(needs_layout_passes=False),
        out_type=[
            jax.ShapeDtypeStruct((A,), jnp.int32),
            jax.ShapeDtypeStruct((P,), jnp.int32),
            jax.ShapeDtypeStruct((P,), jnp.float32),
            jax.ShapeDtypeStruct((NTE,), jnp.int32),
        ],
        scratch_types=[
            pltpu.VMEM((A,), jnp.int32),
            pltpu.VMEM((A,), jnp.float32),
            pltpu.VMEM((A,), jnp.int32),
            pltpu.VMEM((P,), jnp.int32),
            pltpu.VMEM((P,), jnp.float32),
            pltpu.VMEM((SC_L,), jnp.int32),
            pltpu.VMEM((NTE,), jnp.int32),
        ],
    )
    def k(ea_hbm, wa_hbm, pos_hbm, tok_hbm, ws_hbm, te_hbm,
          ea_v, wa_v, pos_v, tok_v, ws_v, cnt_v, te_v):
        wid = lax.axis_index("s") * 2 + lax.axis_index("c")

        @pl.when(wid == 0)
        def _():
            pltpu.sync_copy(ea_hbm, ea_v)
            pltpu.sync_copy(wa_hbm, wa_v)
            lanes = lax.iota(jnp.int32, SC_L)
            zi = jnp.zeros((SC_L,), jnp.int32)
            zf = jnp.zeros((SC_L,), jnp.float32)

            def zinit(i, _):
                tok_v[pl.ds(i * SC_L, SC_L)] = zi
                ws_v[pl.ds(i * SC_L, SC_L)] = zf
                return 0
            lax.fori_loop(0, P // SC_L, zinit, 0)

            cnt_v[...] = zi

            def count_body(c, _):
                ev = ea_v[pl.ds(c * SC_L, SC_L)]
                cnt = cnt_v[...]
                for e in range(E):
                    n = jnp.sum(jnp.where(ev == e, 1, 0))
                    cnt = cnt + jnp.where(lanes == e, n, 0)
                cnt_v[...] = cnt
                return 0
            lax.fori_loop(0, NCH, count_body, 0)

            tot = cnt_v[...]
            padded = (tot + (BMD - 1)) & (-BMD)
            cum = jnp.cumsum(padded)
            offs = cum - padded
            for v in range(NTE // SC_L):
                tiles = (v * SC_L + lanes) * BMD
                tev = zi
                for e in range(E):
                    cum_e = jnp.sum(jnp.where(lanes == e, cum, 0))
                    tev = tev + jnp.where(tiles >= cum_e, 1, 0)
                te_v[pl.ds(v * SC_L, SC_L)] = jnp.minimum(tev, E - 1)
            cnt_v[...] = offs

            def pos_body(c, _):
                av = c * SC_L + lanes
                ev = ea_v[pl.ds(c * SC_L, SC_L)]
                cnt = cnt_v[...]
                pos = zi
                newcnt = cnt
                for e in range(E):
                    m = ev == e
                    base = jnp.sum(jnp.where(lanes == e, cnt, 0))
                    r = jnp.cumsum(m.astype(jnp.int32))
                    pos = jnp.where(m, base + r - 1, pos)
                    n = jnp.max(r)
                    newcnt = newcnt + jnp.where(lanes == e, n, 0)
                cnt_v[...] = newcnt
                pos_v[pl.ds(c * SC_L, SC_L)] = pos
                tok = jnp.where(av >= T, av - T, av)
                plsc.store_scatter(tok_v, [pos], tok)
                plsc.store_scatter(ws_v, [pos], wa_v[pl.ds(c * SC_L, SC_L)])
                return 0
            lax.fori_loop(0, NCH, pos_body, 0)

            pltpu.sync_copy(pos_v, pos_hbm)
            pltpu.sync_copy(tok_v, tok_hbm)
            pltpu.sync_copy(ws_v, ws_hbm)
            pltpu.sync_copy(te_v, te_hbm)

    return k(ea, wa)


def _sc_gather(x, tok, P, CH=16):
    T, D = x.shape
    mesh = plsc.VectorSubcoreMesh(core_axis_name="c", subcore_axis_name="s")
    NW = 32
    per_w = P // NW

    @functools.partial(
        pl.kernel, mesh=mesh,
        compiler_params=pltpu.CompilerParams(needs_layout_passes=False),
        out_type=jax.ShapeDtypeStruct((P, D), jnp.float32),
        scratch_types=[
            pltpu.VMEM((CH,), jnp.int32),
            pltpu.VMEM((CH,), jnp.int32),
            pltpu.VMEM((CH, D), jnp.float32),
            pltpu.VMEM((CH, D), jnp.float32),
            pltpu.SemaphoreType.DMA,
            pltpu.SemaphoreType.DMA,
        ],
    )
    def k(x_hbm, tok_hbm, xs_hbm, idx0, idx1, buf0, buf1, sem0, sem1):
        wid = lax.axis_index("s") * 2 + lax.axis_index("c")
        base = wid * per_w
        idxs = (idx0, idx1)
        bufs = (buf0, buf1)
        sems = (sem0, sem1)
        n = per_w // CH
        pltpu.sync_copy(tok_hbm.at[pl.ds(base, CH)], idx0)
        cps = [pltpu.async_copy(x_hbm.at[idx0], buf0, sem0)]
        for c in range(n):
            b = c % 2
            nb = (c + 1) % 2
            if c + 1 < n:
                pltpu.sync_copy(tok_hbm.at[pl.ds(base + (c + 1) * CH, CH)], idxs[nb])
                cps.append(pltpu.async_copy(x_hbm.at[idxs[nb]], bufs[nb], sems[nb]))
            cps[c].wait()
            pltpu.sync_copy(bufs[b], xs_hbm.at[pl.ds(base + c * CH, CH)])

    return k(x, tok)


def _sc_combine(y, pos, T, CH=16):
    P, D = y.shape
    mesh = plsc.VectorSubcoreMesh(core_axis_name="c", subcore_axis_name="s")
    NW = 32
    per_w = T // NW

    @functools.partial(
        pl.kernel, mesh=mesh,
        compiler_params=pltpu.CompilerParams(needs_layout_passes=False),
        out_type=[
            jax.ShapeDtypeStruct((T, D), jnp.float32),
            jax.ShapeDtypeStruct((T, D), jnp.float32),
        ],
        scratch_types=[
            pltpu.VMEM((CH,), jnp.int32),
            pltpu.VMEM((CH,), jnp.int32),
            pltpu.VMEM((CH, D), jnp.float32),
            pltpu.VMEM((CH, D), jnp.float32),
            pltpu.SemaphoreType.DMA,
            pltpu.SemaphoreType.DMA,
        ],
    )
    def k(y_hbm, pos_hbm, m0_hbm, m1_hbm, idx0, idx1, buf0, buf1, sem0, sem1):
        wid = lax.axis_index("s") * 2 + lax.axis_index("c")
        base = wid * per_w
        nch = per_w // CH
        idxs = (idx0, idx1)
        bufs = (buf0, buf1)
        sems = (sem0, sem1)
        outs = (m0_hbm, m1_hbm)

        def src(i):
            kk, c = divmod(i, nch)
            return pl.ds(kk * T + base + c * CH, CH)

        def dst(i):
            kk, c = divmod(i, nch)
            return outs[kk].at[pl.ds(base + c * CH, CH)]

        n = 2 * nch
        pltpu.sync_copy(pos_hbm.at[src(0)], idx0)
        cps = [pltpu.async_copy(y_hbm.at[idx0], buf0, sem0)]
        for i in range(n):
            b = i % 2
            nb = (i + 1) % 2
            if i + 1 < n:
                pltpu.sync_copy(pos_hbm.at[src(i + 1)], idxs[nb])
                cps.append(pltpu.async_copy(y_hbm.at[idxs[nb]], bufs[nb], sems[nb]))
            cps[i].wait()
            pltpu.sync_copy(bufs[b], dst(i))

    return k(y, pos)


def _s1_body(x_ref, wg_ref, wu_ref, o_ref):
    x = x_ref[...]
    g = _dot(x, wg_ref[...])
    u = _dot(x, wu_ref[...])
    o_ref[...] = (g * jax.nn.sigmoid(g)) * u


def _shared1(x, sw1, bm, bn, s0, nb):
    T, D = x.shape
    SFF = sw1.shape[1] // 2
    nsh = SFF // bn
    return pl.pallas_call(
        _s1_body,
        grid=(nb, T // bm),
        in_specs=[
            pl.BlockSpec((bm, D), lambda n, m: (m, 0)),
            pl.BlockSpec((D, bn), lambda n, m: (0, s0 + n)),
            pl.BlockSpec((D, bn), lambda n, m: (0, nsh + s0 + n)),
        ],
        out_specs=pl.BlockSpec((bm, bn), lambda n, m: (m, n)),
        out_shape=jax.ShapeDtypeStruct((T, nb * bn), F32),
    )(x, sw1, sw1)


def _s2_body(ha_ref, hb_ref, wa_ref, wb_ref, h1_ref, m0_ref, m1_ref,
             sg_ref, o_ref):
    y = _dot(ha_ref[...], wa_ref[...]) + _dot(hb_ref[...], wb_ref[...])
    o_ref[...] = h1_ref[...] + m0_ref[...] + m1_ref[...] + sg_ref[...] * y


def _shared2(hsa, hsb, sw2, h1, m0, m1, sg, bm, bn):
    T, SFFA = hsa.shape
    D = sw2.shape[1]
    return pl.pallas_call(
        _s2_body,
        grid=(D // bn, T // bm),
        in_specs=[
            pl.BlockSpec((bm, SFFA), lambda n, m: (m, 0)),
            pl.BlockSpec((bm, SFFA), lambda n, m: (m, 0)),
            pl.BlockSpec((SFFA, bn), lambda n, m: (0, n)),
            pl.BlockSpec((SFFA, bn), lambda n, m: (1, n)),
            pl.BlockSpec((bm, bn), lambda n, m: (m, n)),
            pl.BlockSpec((bm, bn), lambda n, m: (m, n)),
            pl.BlockSpec((bm, bn), lambda n, m: (m, n)),
            pl.BlockSpec((bm, 1), lambda n, m: (m, 0)),
        ],
        out_specs=pl.BlockSpec((bm, bn), lambda n, m: (m, n)),
        out_shape=jax.ShapeDtypeStruct((T, D), F32),
    )(hsa, hsb, sw2, sw2, h1, m0, m1, sg)


def kernel(hidden_states, positions, ln1_w, ln2_w, wqkv, bqkv, wo, gate_w, w1, w2, sw1, sw2, sgate_w):
    T, D = hidden_states.shape
    QKVN = wqkv.shape[1]
    H = wo.shape[0] // HD
    NKV = (QKVN // HD - H) // 2
    E = gate_w.shape[1]
    FF = w2.shape[1]
    SFF = sw2.shape[0]
    K = 2
    A = T * K
    BMD = 128
    P = A + E * BMD
    NT = P // BMD

    inv_freq = 1.0 / (THETA ** (jnp.arange(0, HD, 2, dtype=F32) / HD))
    freqs = positions.astype(F32)[:, None] * inv_freq
    cos = jnp.concatenate([jnp.cos(freqs)] * 2, axis=-1)
    sin = jnp.concatenate([jnp.sin(freqs)] * 2, axis=-1)

    bmr = min(256, T)
    kv_cols = (H + NKV) * HD
    import math
    bnq = _pick(math.gcd(QKVN, kv_cols), 1024)
    qkv = _qkv(hidden_states, ln1_w, wqkv, bqkv, cos, sin,
               kv_cols=kv_cols, bm=bmr, bn=bnq)
    attn = _attention(qkv, T, H, NKV, bq=min(512, T))
    h1 = _wo_proj(attn, wo, hidden_states, bm=bmr, bn=_pick(D, 512))

    gpad = jnp.concatenate(
        [gate_w, sgate_w, jnp.zeros((D, 128 - E - 1), F32)], axis=1)
    x, wi, wf = _router(h1, ln2_w, gpad, E, bm=bmr)
    topi = wi[:, :K]
    topv = wf[:, :K]
    sg = wf[:, 2:3]

    ea = jnp.concatenate([wi[:, 0], wi[:, 1]])
    wa = jnp.concatenate([wf[:, 0], wf[:, 1]])
    NTE = ((NT + SC_L - 1) // SC_L) * SC_L
    pos, tok, ws, te_pad = _sc_meta(ea, wa, T, E, P, BMD, NTE)
    te = te_pad[:NT]
    xs = _sc_gather(x, tok, P)

    SFH = SFF // 2
    bns = _pick(SFH, 1408)
    nbh = SFH // bns
    hsa = _shared1(x, sw1, bm=bmr, bn=bns, s0=0, nb=nbh)
    xs, hsa = jax.lax.optimization_barrier((xs, hsa))

    act = _moe_w1(xs, w1, te, bm=BMD)
    y = _moe_w2(act, w2, ws, te, bm=BMD)
    y, x2 = jax.lax.optimization_barrier((y, x))
    m0, m1 = _sc_combine(y, pos, T)
    hsb = _shared1(x2, sw1, bm=bmr, bn=bns, s0=nbh, nb=nbh)

    out = _shared2(hsa, hsb, sw2, h1, m0, m1, sg, bm=bmr, bn=_pick(D, 512))
    return out


# speedup vs baseline: 1.1893x; 1.0303x over previous
import functools

import jax
import jax.numpy as jnp
from jax import lax
from jax.experimental import pallas as pl
from jax.experimental.pallas import tpu as pltpu
from jax.experimental.pallas import tpu_sc as plsc

EPS = 1e-6
THETA = 10000.0
NEG = -1e30
HD = 128
BF16 = jnp.bfloat16


def _dot(a, b):
    return jnp.dot(a.astype(BF16), b.astype(BF16), preferred_element_type=F32)
F32 = jnp.float32


def _pick(n, pref):
    b = pref
    while n % b:
        b -= 128
    return b


def _rms(x, w):
    return x * jax.lax.rsqrt(jnp.mean(x * x, axis=-1, keepdims=True) + EPS) * w


def _qkv_body(x_ref, ln_ref, w_ref, b_ref, cos_ref, sin_ref, o_ref, *, bn, kv_end):
    x = x_ref[...]
    xn = _rms(x, ln_ref[...])
    o = _dot(xn, w_ref[...]) + b_ref[...]
    n = pl.program_id(1)
    bm = o.shape[0]
    oh = o.reshape(bm, bn // HD, HD)
    o1 = oh[:, :, : HD // 2]
    o2 = oh[:, :, HD // 2 :]
    cos = cos_ref[...][:, None, :]
    sin = sin_ref[...][:, None, :]
    roped = (oh * cos + jnp.concatenate([-o2, o1], axis=-1) * sin).reshape(bm, bn)
    o_ref[...] = jnp.where(n < kv_end, roped, o)


def _qkv(x, ln1, wqkv, bqkv, cos, sin, kv_cols, bm, bn):
    T, D = x.shape
    N = wqkv.shape[1]
    body = functools.partial(_qkv_body, bn=bn, kv_end=kv_cols // bn)
    return pl.pallas_call(
        body,
        grid=(T // bm, N // bn),
        in_specs=[
            pl.BlockSpec((bm, D), lambda m, n: (m, 0)),
            pl.BlockSpec((1, D), lambda m, n: (0, 0)),
            pl.BlockSpec((D, bn), lambda m, n: (0, n)),
            pl.BlockSpec((1, bn), lambda m, n: (0, n)),
            pl.BlockSpec((bm, HD), lambda m, n: (m, 0)),
            pl.BlockSpec((bm, HD), lambda m, n: (m, 0)),
        ],
        out_specs=pl.BlockSpec((bm, bn), lambda m, n: (m, n)),
        out_shape=jax.ShapeDtypeStruct((T, N), F32),
    )(x, ln1.reshape(1, D), wqkv, bqkv.reshape(1, N), cos, sin)


def _attn_body(q_ref, k_ref, v_ref, o_ref, *, bq, scale):
    q = q_ref[...]
    k = k_ref[...]
    s = jax.lax.dot_general(q.astype(BF16), k.astype(BF16), (((1,), (1,)), ((), ())),
                            preferred_element_type=F32) * scale
    Tk = k.shape[0]
    row = pl.program_id(1) * bq + jax.lax.broadcasted_iota(jnp.int32, (bq, Tk), 0)
    col = jax.lax.broadcasted_iota(jnp.int32, (bq, Tk), 1)
    s = jnp.where(row >= col, s, NEG)
    m = jnp.max(s, axis=-1, keepdims=True)
    p = jnp.exp(s - m)
    l = jnp.sum(p, axis=-1, keepdims=True)
    o = _dot(p, v_ref[...])
    o_ref[...] = o / l


def _attention(qkv, T, H, NKV, bq):
    scale = HD ** -0.5
    body = functools.partial(_attn_body, bq=bq, scale=scale)
    return pl.pallas_call(
        body,
        grid=(H, T // bq),
        in_specs=[
            pl.BlockSpec((bq, HD), lambda h, mq: (mq, h)),
            pl.BlockSpec((T, HD), lambda h, mq: (0, H + h)),
            pl.BlockSpec((T, HD), lambda h, mq: (0, H + NKV + h)),
        ],
        out_specs=pl.BlockSpec((bq, HD), lambda h, mq: (mq, h)),
        out_shape=jax.ShapeDtypeStruct((T, H * HD), F32),
    )(qkv, qkv, qkv)


def _wo_body(a_ref, w_ref, r_ref, o_ref):
    o_ref[...] = r_ref[...] + _dot(a_ref[...], w_ref[...])


def _wo_proj(attn, wo, resid, bm, bn):
    T, Din = attn.shape
    D = wo.shape[1]
    return pl.pallas_call(
        _wo_body,
        grid=(T // bm, D // bn),
        in_specs=[
            pl.BlockSpec((bm, Din), lambda m, n: (m, 0)),
            pl.BlockSpec((Din, bn), lambda m, n: (0, n)),
            pl.BlockSpec((bm, bn), lambda m, n: (m, n)),
        ],
        out_specs=pl.BlockSpec((bm, bn), lambda m, n: (m, n)),
        out_shape=jax.ShapeDtypeStruct((T, D), F32),
    )(attn, wo, resid)


def _router_body(h_ref, ln_ref, g_ref, x_ref, wi_ref, wf_ref, *, E):
    h = h_ref[...]
    xn = _rms(h, ln_ref[...])
    x_ref[...] = xn
    lg = _dot(xn, g_ref[...])
    bm, C = lg.shape
    cols = jax.lax.broadcasted_iota(jnp.int32, (bm, C), 1)
    rl = jnp.where(cols < E, lg, NEG)
    mx = jnp.max(rl, axis=-1, keepdims=True)
    ex = jnp.exp(rl - mx)
    pr = ex / jnp.sum(ex, axis=-1, keepdims=True)
    v1 = jnp.max(pr, axis=-1, keepdims=True)
    i1 = jnp.min(jnp.where(pr == v1, cols, C), axis=-1, keepdims=True)
    pr2 = jnp.where(cols == i1, -1.0, pr)
    v2 = jnp.max(pr2, axis=-1, keepdims=True)
    i2 = jnp.min(jnp.where(pr2 == v2, cols, C), axis=-1, keepdims=True)
    wsum = v1 + v2
    sgl = jnp.sum(jnp.where(cols == E, lg, 0.0), axis=-1, keepdims=True)
    sg = jax.nn.sigmoid(sgl)
    wi_ref[...] = jnp.where(cols == 0, i1, jnp.where(cols == 1, i2, 0))
    wf_ref[...] = jnp.where(cols == 0, v1 / wsum,
                            jnp.where(cols == 1, v2 / wsum,
                                      jnp.where(cols == 2, sg, 0.0)))


def _router(h1, ln2, gpad, E, bm):
    T, D = h1.shape
    C = gpad.shape[1]
    body = functools.partial(_router_body, E=E)
    return pl.pallas_call(
        body,
        grid=(T // bm,),
        in_specs=[
            pl.BlockSpec((bm, D), lambda m: (m, 0)),
            pl.BlockSpec((1, D), lambda m: (0, 0)),
            pl.BlockSpec((D, C), lambda m: (0, 0)),
        ],
        out_specs=[
            pl.BlockSpec((bm, D), lambda m: (m, 0)),
            pl.BlockSpec((bm, C), lambda m: (m, 0)),
            pl.BlockSpec((bm, C), lambda m: (m, 0)),
        ],
        out_shape=[
            jax.ShapeDtypeStruct((T, D), F32),
            jax.ShapeDtypeStruct((T, C), jnp.int32),
            jax.ShapeDtypeStruct((T, C), F32),
        ],
    )(h1, ln2.reshape(1, D), gpad)


def _w1_body(te_ref, xs_ref, wg_ref, wu_ref, o_ref):
    x = xs_ref[...]
    D, FF = wg_ref.shape[1], wg_ref.shape[2]
    wg = wg_ref[...].reshape(D, FF)
    wu = wu_ref[...].reshape(D, FF)
    g = _dot(x, wg)
    u = _dot(x, wu)
    o_ref[...] = (g * jax.nn.sigmoid(g)) * u


def _moe_w1(xs, w1, te, bm):
    P, D = xs.shape
    E = w1.shape[0]
    FF = w1.shape[2] // 2
    grid_spec = pltpu.PrefetchScalarGridSpec(
        num_scalar_prefetch=1,
        grid=(P // bm,),
        in_specs=[
            pl.BlockSpec((bm, D), lambda m, te: (m, 0)),
            pl.BlockSpec((1, D, FF), lambda m, te: (te[m], 0, 0)),
            pl.BlockSpec((1, D, FF), lambda m, te: (te[m], 0, 1)),
        ],
        out_specs=pl.BlockSpec((bm, FF), lambda m, te: (m, 0)),
    )
    return pl.pallas_call(
        _w1_body,
        grid_spec=grid_spec,
        out_shape=jax.ShapeDtypeStruct((P, FF), F32),
    )(te, xs, w1, w1)


def _w2_body(te_ref, a_ref, w_ref, ws_ref, o_ref):
    a = a_ref[...]
    FF, D = w_ref.shape[1], w_ref.shape[2]
    w = w_ref[...].reshape(FF, D)
    y = _dot(a, w)
    o_ref[...] = y * ws_ref[...]


def _moe_w2(act, w2, ws, te, bm):
    P, FF = act.shape
    E, _, D = w2.shape
    grid_spec = pltpu.PrefetchScalarGridSpec(
        num_scalar_prefetch=1,
        grid=(P // bm,),
        in_specs=[
            pl.BlockSpec((bm, FF), lambda m, te: (m, 0)),
            pl.BlockSpec((1, FF, D), lambda m, te: (te[m], 0, 0)),
            pl.BlockSpec((bm, 1), lambda m, te: (m, 0)),
        ],
        out_specs=pl.BlockSpec((bm, D), lambda m, te: (m, 0)),
    )
    return pl.pallas_call(
        _w2_body,
        grid_spec=grid_spec,
        out_shape=jax.ShapeDtypeStruct((P, D), F32),
    )(te, act, w2, ws.reshape(P, 1))


SC_L = 16


def _sc_meta(ea, wa, T, E, P, BMD, NTE):
    A = ea.shape[0]
    NCH = A // SC_L
    mesh = plsc.VectorSubcoreMesh(core_axis_name="c", subcore_axis_name="s")

    @functools.partial(
        pl.kernel, mesh=mesh,
        compiler_params=pltpu.CompilerParams(needs_layout_passes=False),
        out_type=[
            jax.ShapeDtypeStruct((A,), jnp.int32),
            jax.ShapeDtypeStruct((P,), jnp.int32),
            jax.ShapeDtypeStruct((P,), jnp.float32),
            jax.ShapeDtypeStruct((NTE,), jnp.int32),
        ],
        scratch_types=[
            pltpu.VMEM((A,), jnp.int32),
            pltpu.VMEM((A,), jnp.float32),
            pltpu.VMEM((A,), jnp.int32),
            pltpu.VMEM((P,), jnp.int32),
            pltpu.VMEM((P,), jnp.float32),
            pltpu.VMEM((SC_L,), jnp.int32),
            pltpu.VMEM((NTE,), jnp.int32),
        ],
    )
    def k(ea_hbm, wa_hbm, pos_hbm, tok_hbm, ws_hbm, te_hbm,
          ea_v, wa_v, pos_v, tok_v, ws_v, cnt_v, te_v):
        wid = lax.axis_index("s") * 2 + lax.axis_index("c")

        @pl.when(wid == 0)
        def _():
            pltpu.sync_copy(ea_hbm, ea_v)
            pltpu.sync_copy(wa_hbm, wa_v)
            lanes = lax.iota(jnp.int32, SC_L)
            zi = jnp.zeros((SC_L,), jnp.int32)
            zf = jnp.zeros((SC_L,), jnp.float32)

            def zinit(i, _):
                tok_v[pl.ds(i * SC_L, SC_L)] = zi
                ws_v[pl.ds(i * SC_L, SC_L)] = zf
                return 0
            lax.fori_loop(0, P // SC_L, zinit, 0)

            cnt_v[...] = zi

            def count_body(c, _):
                ev = ea_v[pl.ds(c * SC_L, SC_L)]
                cnt = cnt_v[...]
                for e in range(E):
                    n = jnp.sum(jnp.where(ev == e, 1, 0))
                    cnt = cnt + jnp.where(lanes == e, n, 0)
                cnt_v[...] = cnt
                return 0
            lax.fori_loop(0, NCH, count_body, 0)

            tot = cnt_v[...]
            padded = (tot + (BMD - 1)) & (-BMD)
            cum = jnp.cumsum(padded)
            offs = cum - padded
            for v in range(NTE // SC_L):
                tiles = (v * SC_L + lanes) * BMD
                tev = zi
                for e in range(E):
                    cum_e = jnp.sum(jnp.where(lanes == e, cum, 0))
                    tev = tev + jnp.where(tiles >= cum_e, 1, 0)
                te_v[pl.ds(v * SC_L, SC_L)] = jnp.minimum(tev, E - 1)
            cnt_v[...] = offs

            def pos_body(c, _):
                av = c * SC_L + lanes
                ev = ea_v[pl.ds(c * SC_L, SC_L)]
                cnt = cnt_v[...]
                pos = zi
                newcnt = cnt
                for e in range(E):
                    m = ev == e
                    base = jnp.sum(jnp.where(lanes == e, cnt, 0))
                    r = jnp.cumsum(m.astype(jnp.int32))
                    pos = jnp.where(m, base + r - 1, pos)
                    n = jnp.max(r)
                    newcnt = newcnt + jnp.where(lanes == e, n, 0)
                cnt_v[...] = newcnt
                pos_v[pl.ds(c * SC_L, SC_L)] = pos
                tok = jnp.where(av >= T, av - T, av)
                plsc.store_scatter(tok_v, [pos], tok)
                plsc.store_scatter(ws_v, [pos], wa_v[pl.ds(c * SC_L, SC_L)])
                return 0
            lax.fori_loop(0, NCH, pos_body, 0)

            pltpu.sync_copy(pos_v, pos_hbm)
            pltpu.sync_copy(tok_v, tok_hbm)
            pltpu.sync_copy(ws_v, ws_hbm)
            pltpu.sync_copy(te_v, te_hbm)

    return k(ea, wa)


def _sc_gather(x, tok, P, CH=16):
    T, D = x.shape
    mesh = plsc.VectorSubcoreMesh(core_axis_name="c", subcore_axis_name="s")
    NW = 32
    per_w = P // NW

    @functools.partial(
        pl.kernel, mesh=mesh,
        compiler_params=pltpu.CompilerParams(needs_layout_passes=False),
        out_type=jax.ShapeDtypeStruct((P, D), jnp.float32),
        scratch_types=[
            pltpu.VMEM((CH,), jnp.int32),
            pltpu.VMEM((CH,), jnp.int32),
            pltpu.VMEM((CH, D), jnp.float32),
            pltpu.VMEM((CH, D), jnp.float32),
            pltpu.SemaphoreType.DMA,
            pltpu.SemaphoreType.DMA,
        ],
    )
    def k(x_hbm, tok_hbm, xs_hbm, idx0, idx1, buf0, buf1, sem0, sem1):
        wid = lax.axis_index("s") * 2 + lax.axis_index("c")
        base = wid * per_w
        idxs = (idx0, idx1)
        bufs = (buf0, buf1)
        sems = (sem0, sem1)
        n = per_w // CH
        pltpu.sync_copy(tok_hbm.at[pl.ds(base, CH)], idx0)
        cps = [pltpu.async_copy(x_hbm.at[idx0], buf0, sem0)]
        for c in range(n):
            b = c % 2
            nb = (c + 1) % 2
            if c + 1 < n:
                pltpu.sync_copy(tok_hbm.at[pl.ds(base + (c + 1) * CH, CH)], idxs[nb])
                cps.append(pltpu.async_copy(x_hbm.at[idxs[nb]], bufs[nb], sems[nb]))
            cps[c].wait()
            pltpu.sync_copy(bufs[b], xs_hbm.at[pl.ds(base + c * CH, CH)])

    return k(x, tok)


def _sc_combine(y, pos, T, CH=16):
    P, D = y.shape
    mesh = plsc.VectorSubcoreMesh(core_axis_name="c", subcore_axis_name="s")
    NW = 32
    per_w = T // NW

    @functools.partial(
        pl.kernel, mesh=mesh,
        compiler_params=pltpu.CompilerParams(needs_layout_passes=False),
        out_type=[
            jax.ShapeDtypeStruct((T, D), jnp.float32),
            jax.ShapeDtypeStruct((T, D), jnp.float32),
        ],
        scratch_types=[
            pltpu.VMEM((CH,), jnp.int32),
            pltpu.VMEM((CH,), jnp.int32),
            pltpu.VMEM((CH, D), jnp.float32),
            pltpu.VMEM((CH, D), jnp.float32),
            pltpu.SemaphoreType.DMA,
            pltpu.SemaphoreType.DMA,
        ],
    )
    def k(y_hbm, pos_hbm, m0_hbm, m1_hbm, idx0, idx1, buf0, buf1, sem0, sem1):
        wid = lax.axis_index("s") * 2 + lax.axis_index("c")
        base = wid * per_w
        nch = per_w // CH
        idxs = (idx0, idx1)
        bufs = (buf0, buf1)
        sems = (sem0, sem1)
        outs = (m0_hbm, m1_hbm)

        def src(i):
            kk, c = divmod(i, nch)
            return pl.ds(kk * T + base + c * CH, CH)

        def dst(i):
            kk, c = divmod(i, nch)
            return outs[kk].at[pl.ds(base + c * CH, CH)]

        n = 2 * nch
        pltpu.sync_copy(pos_hbm.at[src(0)], idx0)
        cps = [pltpu.async_copy(y_hbm.at[idx0], buf0, sem0)]
        for i in range(n):
            b = i % 2
            nb = (i + 1) % 2
            if i + 1 < n:
                pltpu.sync_copy(pos_hbm.at[src(i + 1)], idxs[nb])
                cps.append(pltpu.async_copy(y_hbm.at[idxs[nb]], bufs[nb], sems[nb]))
            cps[i].wait()
            pltpu.sync_copy(bufs[b], dst(i))

    return k(y, pos)


def _s1_body(x_ref, wg_ref, wu_ref, o_ref):
    x = x_ref[...]
    g = _dot(x, wg_ref[...])
    u = _dot(x, wu_ref[...])
    o_ref[...] = (g * jax.nn.sigmoid(g)) * u


def _shared1(x, sw1, bm, bn, s0, nb):
    T, D = x.shape
    SFF = sw1.shape[1] // 2
    nsh = SFF // bn
    return pl.pallas_call(
        _s1_body,
        grid=(nb, T // bm),
        in_specs=[
            pl.BlockSpec((bm, D), lambda n, m: (m, 0)),
            pl.BlockSpec((D, bn), lambda n, m: (0, s0 + n)),
            pl.BlockSpec((D, bn), lambda n, m: (0, nsh + s0 + n)),
        ],
        out_specs=pl.BlockSpec((bm, bn), lambda n, m: (m, n)),
        out_shape=jax.ShapeDtypeStruct((T, nb * bn), F32),
    )(x, sw1, sw1)


def _s2_body(ha_ref, hb_ref, wa_ref, wb_ref, h1_ref, m0_ref, m1_ref,
             sg_ref, o_ref):
    y = _dot(ha_ref[...], wa_ref[...]) + _dot(hb_ref[...], wb_ref[...])
    o_ref[...] = h1_ref[...] + m0_ref[...] + m1_ref[...] + sg_ref[...] * y


def _shared2(hsa, hsb, sw2, h1, m0, m1, sg, bm, bn):
    T, SFFA = hsa.shape
    D = sw2.shape[1]
    return pl.pallas_call(
        _s2_body,
        grid=(D // bn, T // bm),
        in_specs=[
            pl.BlockSpec((bm, SFFA), lambda n, m: (m, 0)),
            pl.BlockSpec((bm, SFFA), lambda n, m: (m, 0)),
            pl.BlockSpec((SFFA, bn), lambda n, m: (0, n)),
            pl.BlockSpec((SFFA, bn), lambda n, m: (1, n)),
            pl.BlockSpec((bm, bn), lambda n, m: (m, n)),
            pl.BlockSpec((bm, bn), lambda n, m: (m, n)),
            pl.BlockSpec((bm, bn), lambda n, m: (m, n)),
            pl.BlockSpec((bm, 1), lambda n, m: (m, 0)),
        ],
        out_specs=pl.BlockSpec((bm, bn), lambda n, m: (m, n)),
        out_shape=jax.ShapeDtypeStruct((T, D), F32),
    )(hsa, hsb, sw2, sw2, h1, m0, m1, sg)


def kernel(hidden_states, positions, ln1_w, ln2_w, wqkv, bqkv, wo, gate_w, w1, w2, sw1, sw2, sgate_w):
    T, D = hidden_states.shape
    QKVN = wqkv.shape[1]
    H = wo.shape[0] // HD
    NKV = (QKVN // HD - H) // 2
    E = gate_w.shape[1]
    FF = w2.shape[1]
    SFF = sw2.shape[0]
    K = 2
    A = T * K
    BMD = 128
    P = A + E * BMD
    NT = P // BMD

    inv_freq = 1.0 / (THETA ** (jnp.arange(0, HD, 2, dtype=F32) / HD))
    freqs = positions.astype(F32)[:, None] * inv_freq
    cos = jnp.concatenate([jnp.cos(freqs)] * 2, axis=-1)
    sin = jnp.concatenate([jnp.sin(freqs)] * 2, axis=-1)

    bmr = min(256, T)
    kv_cols = (H + NKV) * HD
    import math
    bnq = _pick(math.gcd(QKVN, kv_cols), 1024)
    qkv = _qkv(hidden_states, ln1_w, wqkv, bqkv, cos, sin,
               kv_cols=kv_cols, bm=bmr, bn=bnq)
    attn = _attention(qkv, T, H, NKV, bq=bmr)
    h1 = _wo_proj(attn, wo, hidden_states, bm=bmr, bn=_pick(D, 512))

    gpad = jnp.concatenate(
        [gate_w, sgate_w, jnp.zeros((D, 128 - E - 1), F32)], axis=1)
    x, wi, wf = _router(h1, ln2_w, gpad, E, bm=bmr)
    topi = wi[:, :K]
    topv = wf[:, :K]
    sg = wf[:, 2:3]

    ea = jnp.concatenate([wi[:, 0], wi[:, 1]])
    wa = jnp.concatenate([wf[:, 0], wf[:, 1]])
    NTE = ((NT + SC_L - 1) // SC_L) * SC_L
    pos, tok, ws, te_pad = _sc_meta(ea, wa, T, E, P, BMD, NTE)
    te = te_pad[:NT]
    xs = _sc_gather(x, tok, P)

    SFH = SFF // 2
    bns = _pick(SFH, 1408)
    nbh = SFH // bns
    hsa = _shared1(x, sw1, bm=bmr, bn=bns, s0=0, nb=nbh)
    xs, hsa = jax.lax.optimization_barrier((xs, hsa))

    act = _moe_w1(xs, w1, te, bm=BMD)
    y = _moe_w2(act, w2, ws, te, bm=BMD)
    y, x2 = jax.lax.optimization_barrier((y, x))
    m0, m1 = _sc_combine(y, pos, T)
    hsb = _shared1(x2, sw1, bm=bmr, bn=bns, s0=nbh, nb=nbh)

    out = _shared2(hsa, hsb, sw2, h1, m0, m1, sg, bm=bmr, bn=_pick(D, 512))
    return out


# speedup vs baseline: 1.1958x; 1.0055x over previous
import functools

import jax
import jax.numpy as jnp
from jax import lax
from jax.experimental import pallas as pl
from jax.experimental.pallas import tpu as pltpu
from jax.experimental.pallas import tpu_sc as plsc

EPS = 1e-6
THETA = 10000.0
NEG = -1e30
HD = 128
BF16 = jnp.bfloat16


def _dot(a, b):
    return jnp.dot(a.astype(BF16), b.astype(BF16), preferred_element_type=F32)
F32 = jnp.float32


def _pick(n, pref):
    b = pref
    while n % b:
        b -= 128
    return b


def _rms(x, w):
    return x * jax.lax.rsqrt(jnp.mean(x * x, axis=-1, keepdims=True) + EPS) * w


def _qkv_body(x_ref, ln_ref, w_ref, b_ref, cos_ref, sin_ref, o_ref, *, bn, kv_end):
    x = x_ref[...]
    xn = _rms(x, ln_ref[...])
    o = _dot(xn, w_ref[...]) + b_ref[...]
    n = pl.program_id(1)
    bm = o.shape[0]
    oh = o.reshape(bm, bn // HD, HD)
    o1 = oh[:, :, : HD // 2]
    o2 = oh[:, :, HD // 2 :]
    cos = cos_ref[...][:, None, :]
    sin = sin_ref[...][:, None, :]
    roped = (oh * cos + jnp.concatenate([-o2, o1], axis=-1) * sin).reshape(bm, bn)
    o_ref[...] = jnp.where(n < kv_end, roped, o)


def _qkv(x, ln1, wqkv, bqkv, cos, sin, kv_cols, bm, bn):
    T, D = x.shape
    N = wqkv.shape[1]
    body = functools.partial(_qkv_body, bn=bn, kv_end=kv_cols // bn)
    return pl.pallas_call(
        body,
        grid=(T // bm, N // bn),
        in_specs=[
            pl.BlockSpec((bm, D), lambda m, n: (m, 0)),
            pl.BlockSpec((1, D), lambda m, n: (0, 0)),
            pl.BlockSpec((D, bn), lambda m, n: (0, n)),
            pl.BlockSpec((1, bn), lambda m, n: (0, n)),
            pl.BlockSpec((bm, HD), lambda m, n: (m, 0)),
            pl.BlockSpec((bm, HD), lambda m, n: (m, 0)),
        ],
        out_specs=pl.BlockSpec((bm, bn), lambda m, n: (m, n)),
        out_shape=jax.ShapeDtypeStruct((T, N), F32),
    )(x, ln1.reshape(1, D), wqkv, bqkv.reshape(1, N), cos, sin)


def _attn_body(q_ref, k_ref, v_ref, o_ref, *, bq, scale):
    q = q_ref[...]
    k = k_ref[...]
    s = jax.lax.dot_general(q.astype(BF16), k.astype(BF16), (((1,), (1,)), ((), ())),
                            preferred_element_type=F32) * scale
    Tk = k.shape[0]
    row = pl.program_id(1) * bq + jax.lax.broadcasted_iota(jnp.int32, (bq, Tk), 0)
    col = jax.lax.broadcasted_iota(jnp.int32, (bq, Tk), 1)
    s = jnp.where(row >= col, s, NEG)
    m = jnp.max(s, axis=-1, keepdims=True)
    p = jnp.exp(s - m)
    l = jnp.sum(p, axis=-1, keepdims=True)
    o = _dot(p, v_ref[...])
    o_ref[...] = o / l


def _attention(qkv, T, H, NKV, bq):
    scale = HD ** -0.5
    body = functools.partial(_attn_body, bq=bq, scale=scale)
    return pl.pallas_call(
        body,
        grid=(H, T // bq),
        in_specs=[
            pl.BlockSpec((bq, HD), lambda h, mq: (mq, h)),
            pl.BlockSpec((T, HD), lambda h, mq: (0, H + h)),
            pl.BlockSpec((T, HD), lambda h, mq: (0, H + NKV + h)),
        ],
        out_specs=pl.BlockSpec((bq, HD), lambda h, mq: (mq, h)),
        out_shape=jax.ShapeDtypeStruct((T, H * HD), F32),
    )(qkv, qkv, qkv)


def _wo_body(a_ref, w_ref, r_ref, o_ref):
    o_ref[...] = r_ref[...] + _dot(a_ref[...], w_ref[...])


def _wo_proj(attn, wo, resid, bm, bn):
    T, Din = attn.shape
    D = wo.shape[1]
    return pl.pallas_call(
        _wo_body,
        grid=(T // bm, D // bn),
        in_specs=[
            pl.BlockSpec((bm, Din), lambda m, n: (m, 0)),
            pl.BlockSpec((Din, bn), lambda m, n: (0, n)),
            pl.BlockSpec((bm, bn), lambda m, n: (m, n)),
        ],
        out_specs=pl.BlockSpec((bm, bn), lambda m, n: (m, n)),
        out_shape=jax.ShapeDtypeStruct((T, D), F32),
    )(attn, wo, resid)


def _router_body(h_ref, ln_ref, g_ref, x_ref, wi_ref, wf_ref, *, E):
    h = h_ref[...]
    xn = _rms(h, ln_ref[...])
    x_ref[...] = xn
    lg = _dot(xn, g_ref[...])
    bm, C = lg.shape
    cols = jax.lax.broadcasted_iota(jnp.int32, (bm, C), 1)
    rl = jnp.where(cols < E, lg, NEG)
    mx = jnp.max(rl, axis=-1, keepdims=True)
    ex = jnp.exp(rl - mx)
    pr = ex / jnp.sum(ex, axis=-1, keepdims=True)
    v1 = jnp.max(pr, axis=-1, keepdims=True)
    i1 = jnp.min(jnp.where(pr == v1, cols, C), axis=-1, keepdims=True)
    pr2 = jnp.where(cols == i1, -1.0, pr)
    v2 = jnp.max(pr2, axis=-1, keepdims=True)
    i2 = jnp.min(jnp.where(pr2 == v2, cols, C), axis=-1, keepdims=True)
    wsum = v1 + v2
    sgl = jnp.sum(jnp.where(cols == E, lg, 0.0), axis=-1, keepdims=True)
    sg = jax.nn.sigmoid(sgl)
    wi_ref[...] = jnp.where(cols == 0, i1, jnp.where(cols == 1, i2, 0))
    wf_ref[...] = jnp.where(cols == 0, v1 / wsum,
                            jnp.where(cols == 1, v2 / wsum,
                                      jnp.where(cols == 2, sg, 0.0)))


def _router(h1, ln2, gpad, E, bm):
    T, D = h1.shape
    C = gpad.shape[1]
    body = functools.partial(_router_body, E=E)
    return pl.pallas_call(
        body,
        grid=(T // bm,),
        in_specs=[
            pl.BlockSpec((bm, D), lambda m: (m, 0)),
            pl.BlockSpec((1, D), lambda m: (0, 0)),
            pl.BlockSpec((D, C), lambda m: (0, 0)),
        ],
        out_specs=[
            pl.BlockSpec((bm, D), lambda m: (m, 0)),
            pl.BlockSpec((bm, C), lambda m: (m, 0)),
            pl.BlockSpec((bm, C), lambda m: (m, 0)),
        ],
        out_shape=[
            jax.ShapeDtypeStruct((T, D), F32),
            jax.ShapeDtypeStruct((T, C), jnp.int32),
            jax.ShapeDtypeStruct((T, C), F32),
        ],
    )(h1, ln2.reshape(1, D), gpad)


def _w1_body(te_ref, xs_ref, wg_ref, wu_ref, o_ref):
    x = xs_ref[...]
    D, FF = wg_ref.shape[1], wg_ref.shape[2]
    wg = wg_ref[...].reshape(D, FF)
    wu = wu_ref[...].reshape(D, FF)
    g = _dot(x, wg)
    u = _dot(x, wu)
    o_ref[...] = (g * jax.nn.sigmoid(g)) * u


def _moe_w1(xs, w1, te, bm):
    P, D = xs.shape
    E = w1.shape[0]
    FF = w1.shape[2] // 2
    grid_spec = pltpu.PrefetchScalarGridSpec(
        num_scalar_prefetch=1,
        grid=(P // bm,),
        in_specs=[
            pl.BlockSpec((bm, D), lambda m, te: (m, 0)),
            pl.BlockSpec((1, D, FF), lambda m, te: (te[m], 0, 0)),
            pl.BlockSpec((1, D, FF), lambda m, te: (te[m], 0, 1)),
        ],
        out_specs=pl.BlockSpec((bm, FF), lambda m, te: (m, 0)),
    )
    return pl.pallas_call(
        _w1_body,
        grid_spec=grid_spec,
        out_shape=jax.ShapeDtypeStruct((P, FF), F32),
    )(te, xs, w1, w1)


def _w2_body(te_ref, a_ref, w_ref, ws_ref, o_ref):
    a = a_ref[...]
    FF, D = w_ref.shape[1], w_ref.shape[2]
    w = w_ref[...].reshape(FF, D)
    y = _dot(a, w)
    o_ref[...] = y * ws_ref[...]


def _moe_w2(act, w2, ws, te, bm):
    P, FF = act.shape
    E, _, D = w2.shape
    grid_spec = pltpu.PrefetchScalarGridSpec(
        num_scalar_prefetch=1,
        grid=(P // bm,),
        in_specs=[
            pl.BlockSpec((bm, FF), lambda m, te: (m, 0)),
            pl.BlockSpec((1, FF, D), lambda m, te: (te[m], 0, 0)),
            pl.BlockSpec((bm, 1), lambda m, te: (m, 0)),
        ],
        out_specs=pl.BlockSpec((bm, D), lambda m, te: (m, 0)),
    )
    return pl.pallas_call(
        _w2_body,
        grid_spec=grid_spec,
        out_shape=jax.ShapeDtypeStruct((P, D), F32),
    )(te, act, w2, ws.reshape(P, 1))


SC_L = 16


def _sc_meta(ea, wa, T, E, P, BMD, NTE):
    A = ea.shape[0]
    NCH = A // SC_L
    mesh = plsc.VectorSubcoreMesh(core_axis_name="c", subcore_axis_name="s")

    @functools.partial(
        pl.kernel, mesh=mesh,
        compiler_params=pltpu.CompilerParams(needs_layout_passes=False),
        out_type=[
            jax.ShapeDtypeStruct((A,), jnp.int32),
            jax.ShapeDtypeStruct((P,), jnp.int32),
            jax.ShapeDtypeStruct((P,), jnp.float32),
            jax.ShapeDtypeStruct((NTE,), jnp.int32),
        ],
        scratch_types=[
            pltpu.VMEM((A,), jnp.int32),
            pltpu.VMEM((A,), jnp.float32),
            pltpu.VMEM((A,), jnp.int32),
            pltpu.VMEM((P,), jnp.int32),
            pltpu.VMEM((P,), jnp.float32),
            pltpu.VMEM((SC_L,), jnp.int32),
            pltpu.VMEM((NTE,), jnp.int32),
        ],
    )
    def k(ea_hbm, wa_hbm, pos_hbm, tok_hbm, ws_hbm, te_hbm,
          ea_v, wa_v, pos_v, tok_v, ws_v, cnt_v, te_v):
        wid = lax.axis_index("s") * 2 + lax.axis_index("c")

        @pl.when(wid == 0)
        def _():
            pltpu.sync_copy(ea_hbm, ea_v)
            pltpu.sync_copy(wa_hbm, wa_v)
            lanes = lax.iota(jnp.int32, SC_L)
            zi = jnp.zeros((SC_L,), jnp.int32)
            zf = jnp.zeros((SC_L,), jnp.float32)

            def zinit(i, _):
                tok_v[pl.ds(i * SC_L, SC_L)] = zi
                ws_v[pl.ds(i * SC_L, SC_L)] = zf
                return 0
            lax.fori_loop(0, P // SC_L, zinit, 0)

            cnt_v[...] = zi

            def count_body(c, _):
                ev = ea_v[pl.ds(c * SC_L, SC_L)]
                cnt = cnt_v[...]
                for e in range(E):
                    n = jnp.sum(jnp.where(ev == e, 1, 0))
                    cnt = cnt + jnp.where(lanes == e, n, 0)
                cnt_v[...] = cnt
                return 0
            lax.fori_loop(0, NCH, count_body, 0)

            tot = cnt_v[...]
            padded = (tot + (BMD - 1)) & (-BMD)
            cum = jnp.cumsum(padded)
            offs = cum - padded
            for v in range(NTE // SC_L):
                tiles = (v * SC_L + lanes) * BMD
                tev = zi
                for e in range(E):
                    cum_e = jnp.sum(jnp.where(lanes == e, cum, 0))
                    tev = tev + jnp.where(tiles >= cum_e, 1, 0)
                te_v[pl.ds(v * SC_L, SC_L)] = jnp.minimum(tev, E - 1)
            cnt_v[...] = offs

            def pos_body(c, _):
                av = c * SC_L + lanes
                ev = ea_v[pl.ds(c * SC_L, SC_L)]
                cnt = cnt_v[...]
                pos = zi
                newcnt = cnt
                for e in range(E):
                    m = ev == e
                    base = jnp.sum(jnp.where(lanes == e, cnt, 0))
                    r = jnp.cumsum(m.astype(jnp.int32))
                    pos = jnp.where(m, base + r - 1, pos)
                    n = jnp.max(r)
                    newcnt = newcnt + jnp.where(lanes == e, n, 0)
                cnt_v[...] = newcnt
                pos_v[pl.ds(c * SC_L, SC_L)] = pos
                tok = jnp.where(av >= T, av - T, av)
                plsc.store_scatter(tok_v, [pos], tok)
                plsc.store_scatter(ws_v, [pos], wa_v[pl.ds(c * SC_L, SC_L)])
                return 0
            lax.fori_loop(0, NCH, pos_body, 0)

            pltpu.sync_copy(pos_v, pos_hbm)
            pltpu.sync_copy(tok_v, tok_hbm)
            pltpu.sync_copy(ws_v, ws_hbm)
            pltpu.sync_copy(te_v, te_hbm)

    return k(ea, wa)


def _sc_gather(x, tok, P, CH=16):
    T, D = x.shape
    mesh = plsc.VectorSubcoreMesh(core_axis_name="c", subcore_axis_name="s")
    NW = 32
    per_w = P // NW

    @functools.partial(
        pl.kernel, mesh=mesh,
        compiler_params=pltpu.CompilerParams(needs_layout_passes=False),
        out_type=jax.ShapeDtypeStruct((P, D), jnp.float32),
        scratch_types=[
            pltpu.VMEM((CH,), jnp.int32),
            pltpu.VMEM((CH,), jnp.int32),
            pltpu.VMEM((CH, D), jnp.float32),
            pltpu.VMEM((CH, D), jnp.float32),
            pltpu.SemaphoreType.DMA,
            pltpu.SemaphoreType.DMA,
        ],
    )
    def k(x_hbm, tok_hbm, xs_hbm, idx0, idx1, buf0, buf1, sem0, sem1):
        wid = lax.axis_index("s") * 2 + lax.axis_index("c")
        base = wid * per_w
        idxs = (idx0, idx1)
        bufs = (buf0, buf1)
        sems = (sem0, sem1)
        n = per_w // CH
        pltpu.sync_copy(tok_hbm.at[pl.ds(base, CH)], idx0)
        cps = [pltpu.async_copy(x_hbm.at[idx0], buf0, sem0)]
        for c in range(n):
            b = c % 2
            nb = (c + 1) % 2
            if c + 1 < n:
                pltpu.sync_copy(tok_hbm.at[pl.ds(base + (c + 1) * CH, CH)], idxs[nb])
                cps.append(pltpu.async_copy(x_hbm.at[idxs[nb]], bufs[nb], sems[nb]))
            cps[c].wait()
            pltpu.sync_copy(bufs[b], xs_hbm.at[pl.ds(base + c * CH, CH)])

    return k(x, tok)


def _sc_combine(y, pos, T, CH=16):
    P, D = y.shape
    mesh = plsc.VectorSubcoreMesh(core_axis_name="c", subcore_axis_name="s")
    NW = 32
    per_w = T // NW

    @functools.partial(
        pl.kernel, mesh=mesh,
        compiler_params=pltpu.CompilerParams(needs_layout_passes=False),
        out_type=[
            jax.ShapeDtypeStruct((T, D), jnp.float32),
            jax.ShapeDtypeStruct((T, D), jnp.float32),
        ],
        scratch_types=[
            pltpu.VMEM((CH,), jnp.int32),
            pltpu.VMEM((CH,), jnp.int32),
            pltpu.VMEM((CH, D), jnp.float32),
            pltpu.VMEM((CH, D), jnp.float32),
            pltpu.SemaphoreType.DMA,
            pltpu.SemaphoreType.DMA,
        ],
    )
    def k(y_hbm, pos_hbm, m0_hbm, m1_hbm, idx0, idx1, buf0, buf1, sem0, sem1):
        wid = lax.axis_index("s") * 2 + lax.axis_index("c")
        base = wid * per_w
        nch = per_w // CH
        idxs = (idx0, idx1)
        bufs = (buf0, buf1)
        sems = (sem0, sem1)
        outs = (m0_hbm, m1_hbm)

        def src(i):
            kk, c = divmod(i, nch)
            return pl.ds(kk * T + base + c * CH, CH)

        def dst(i):
            kk, c = divmod(i, nch)
            return outs[kk].at[pl.ds(base + c * CH, CH)]

        n = 2 * nch
        pltpu.sync_copy(pos_hbm.at[src(0)], idx0)
        cps = [pltpu.async_copy(y_hbm.at[idx0], buf0, sem0)]
        for i in range(n):
            b = i % 2
            nb = (i + 1) % 2
            if i + 1 < n:
                pltpu.sync_copy(pos_hbm.at[src(i + 1)], idxs[nb])
                cps.append(pltpu.async_copy(y_hbm.at[idxs[nb]], bufs[nb], sems[nb]))
            cps[i].wait()
            pltpu.sync_copy(bufs[b], dst(i))

    return k(y, pos)


def _s1_body(x_ref, wg_ref, wu_ref, o_ref):
    x = x_ref[...]
    g = _dot(x, wg_ref[...])
    u = _dot(x, wu_ref[...])
    o_ref[...] = (g * jax.nn.sigmoid(g)) * u


def _shared1(x, sw1, bm, bn, s0, nb):
    T, D = x.shape
    SFF = sw1.shape[1] // 2
    nsh = SFF // bn
    return pl.pallas_call(
        _s1_body,
        grid=(nb, T // bm),
        in_specs=[
            pl.BlockSpec((bm, D), lambda n, m: (m, 0)),
            pl.BlockSpec((D, bn), lambda n, m: (0, s0 + n)),
            pl.BlockSpec((D, bn), lambda n, m: (0, nsh + s0 + n)),
        ],
        out_specs=pl.BlockSpec((bm, bn), lambda n, m: (m, n)),
        out_shape=jax.ShapeDtypeStruct((T, nb * bn), F32),
    )(x, sw1, sw1)


def _s2_body(ha_ref, hb_ref, wa_ref, wb_ref, h1_ref, m0_ref, m1_ref,
             sg_ref, o_ref):
    y = _dot(ha_ref[...], wa_ref[...]) + _dot(hb_ref[...], wb_ref[...])
    o_ref[...] = h1_ref[...] + m0_ref[...] + m1_ref[...] + sg_ref[...] * y


def _shared2(hsa, hsb, sw2, h1, m0, m1, sg, bm, bn):
    T, SFFA = hsa.shape
    D = sw2.shape[1]
    return pl.pallas_call(
        _s2_body,
        grid=(D // bn, T // bm),
        in_specs=[
            pl.BlockSpec((bm, SFFA), lambda n, m: (m, 0)),
            pl.BlockSpec((bm, SFFA), lambda n, m: (m, 0)),
            pl.BlockSpec((SFFA, bn), lambda n, m: (0, n)),
            pl.BlockSpec((SFFA, bn), lambda n, m: (1, n)),
            pl.BlockSpec((bm, bn), lambda n, m: (m, n)),
            pl.BlockSpec((bm, bn), lambda n, m: (m, n)),
            pl.BlockSpec((bm, bn), lambda n, m: (m, n)),
            pl.BlockSpec((bm, 1), lambda n, m: (m, 0)),
        ],
        out_specs=pl.BlockSpec((bm, bn), lambda n, m: (m, n)),
        out_shape=jax.ShapeDtypeStruct((T, D), F32),
    )(hsa, hsb, sw2, sw2, h1, m0, m1, sg)


def kernel(hidden_states, positions, ln1_w, ln2_w, wqkv, bqkv, wo, gate_w, w1, w2, sw1, sw2, sgate_w):
    T, D = hidden_states.shape
    QKVN = wqkv.shape[1]
    H = wo.shape[0] // HD
    NKV = (QKVN // HD - H) // 2
    E = gate_w.shape[1]
    FF = w2.shape[1]
    SFF = sw2.shape[0]
    K = 2
    A = T * K
    BMD = 128
    P = A + E * BMD
    NT = P // BMD

    inv_freq = 1.0 / (THETA ** (jnp.arange(0, HD, 2, dtype=F32) / HD))
    freqs = positions.astype(F32)[:, None] * inv_freq
    cos = jnp.concatenate([jnp.cos(freqs)] * 2, axis=-1)
    sin = jnp.concatenate([jnp.sin(freqs)] * 2, axis=-1)

    bmr = min(256, T)
    kv_cols = (H + NKV) * HD
    import math
    bnq = _pick(math.gcd(QKVN, kv_cols), 1024)
    qkv = _qkv(hidden_states, ln1_w, wqkv, bqkv, cos, sin,
               kv_cols=kv_cols, bm=bmr, bn=bnq)
    attn = _attention(qkv, T, H, NKV, bq=bmr)
    h1 = _wo_proj(attn, wo, hidden_states, bm=bmr, bn=_pick(D, 512))

    gpad = jnp.concatenate(
        [gate_w, sgate_w, jnp.zeros((D, 128 - E - 1), F32)], axis=1)
    x, wi, wf = _router(h1, ln2_w, gpad, E, bm=bmr)
    sg = wf[:, 2:3]

    ea = jnp.concatenate([wi[:, 0], wi[:, 1]])
    wa = jnp.concatenate([wf[:, 0], wf[:, 1]])
    NTE = ((NT + SC_L - 1) // SC_L) * SC_L
    pos, tok, ws, te_pad = _sc_meta(ea, wa, T, E, P, BMD, NTE)
    te = te_pad[:NT]
    xs = _sc_gather(x, tok, P)

    SFH = SFF // 2
    bns = _pick(SFH, 1408)
    nbh = SFH // bns
    hsa = _shared1(x, sw1, bm=bmr, bn=bns, s0=0, nb=nbh)
    xs, hsa = jax.lax.optimization_barrier((xs, hsa))

    act = _moe_w1(xs, w1, te, bm=BMD)
    y = _moe_w2(act, w2, ws, te, bm=BMD)
    y, x2 = jax.lax.optimization_barrier((y, x))
    m0, m1 = _sc_combine(y, pos, T)
    hsb = _shared1(x2, sw1, bm=bmr, bn=bns, s0=nbh, nb=nbh)

    out = _shared2(hsa, hsb, sw2, h1, m0, m1, sg, bm=bmr, bn=_pick(D, 512))
    return out
